# Initial kernel scaffold; baseline (speedup 1.0000x reference)
import jax
import jax.numpy as jnp
from jax.experimental import pallas as pl


def kernel(scal_feat, cat_feat, W, b, table):
    raise NotImplementedError("write your pallas kernel here")


# speedup vs baseline: 5.0407x; 5.0407x over previous
import functools

import jax
import jax.numpy as jnp
from jax import lax
from jax.experimental import pallas as pl
from jax.experimental.pallas import tpu as pltpu
from jax.experimental.pallas import tpu_sc as plsc

VOCAB = 100000
EMBED = 64
N_SCAL = 13
N_CAT = 26
BATCH = 16384

NUM_CORES = 2
NUM_SUBCORES = 16
NUM_WORKERS = NUM_CORES * NUM_SUBCORES
ROWS_PER_W = BATCH // NUM_WORKERS


def _tc_matmul(scal_feat, W, b):
    blk = 2048

    def body(s_ref, w_ref, b_ref, o_ref):
        o_ref[:, :] = (
            jnp.dot(s_ref[:, :], w_ref[:, :], preferred_element_type=jnp.float32)
            + b_ref[:, :]
        )

    return pl.pallas_call(
        body,
        grid=(BATCH // blk,),
        in_specs=[
            pl.BlockSpec((blk, N_SCAL), lambda i: (i, 0)),
            pl.BlockSpec((N_SCAL, EMBED), lambda i: (0, 0)),
            pl.BlockSpec((1, EMBED), lambda i: (0, 0)),
        ],
        out_specs=pl.BlockSpec((blk, EMBED), lambda i: (i, 0)),
        out_shape=jax.ShapeDtypeStruct((BATCH, EMBED), jnp.float32),
    )(scal_feat, W, b.reshape(1, EMBED))


def _sc_assemble(table, cat_t, scal_emb):
    mesh = plsc.VectorSubcoreMesh(core_axis_name="c", subcore_axis_name="s")

    @functools.partial(
        pl.kernel,
        mesh=mesh,
        compiler_params=pltpu.CompilerParams(use_tc_tiling_on_sc=False),
        out_type=jax.ShapeDtypeStruct((BATCH, EMBED * (N_CAT + 1)), jnp.float32),
        scratch_types=[
            pltpu.VMEM((ROWS_PER_W,), jnp.int32),
            pltpu.VMEM((ROWS_PER_W, EMBED), jnp.float32),
            pltpu.VMEM((ROWS_PER_W, EMBED), jnp.float32),
            pltpu.SemaphoreType.DMA,
        ],
    )
    def k(table_hbm, cat_hbm, semb_hbm, out_hbm, idx_v, rows_v, sbuf_v, sem):
        wid = lax.axis_index("s") * NUM_CORES + lax.axis_index("c")
        base = wid * ROWS_PER_W
        pltpu.sync_copy(semb_hbm.at[pl.ds(base, ROWS_PER_W)], sbuf_v)
        pltpu.sync_copy(sbuf_v, out_hbm.at[pl.ds(base, ROWS_PER_W), pl.ds(0, EMBED)])

        def body(c, carry):
            pltpu.sync_copy(cat_hbm.at[c, pl.ds(base, ROWS_PER_W)], idx_v)
            pltpu.async_copy(table_hbm.at[idx_v], rows_v, sem).wait()
            pltpu.sync_copy(
                rows_v,
                out_hbm.at[pl.ds(base, ROWS_PER_W), pl.ds((c + 1) * EMBED, EMBED)],
            )
            return carry

        lax.fori_loop(0, N_CAT, body, 0)

    return k(table, cat_t, scal_emb)


def kernel(scal_feat, cat_feat, W, b, table):
    scal_emb = _tc_matmul(scal_feat, W, b)
    cat_t = cat_feat.astype(jnp.int32).T
    return _sc_assemble(table, cat_t, scal_emb)
